# Initial kernel scaffold; baseline (speedup 1.0000x reference)
import jax
import jax.numpy as jnp
from jax.experimental import pallas as pl


def kernel(x, edge_index, edge_type, W_fc, b_fc, W1, a1_src, a1_dst, W2, a2_src, a2_dst, Wc1, bc1, Wc2, bc2):
    raise NotImplementedError("write your pallas kernel here")


# speedup vs baseline: 12.1592x; 12.1592x over previous
import functools

import jax
import jax.numpy as jnp
from jax import lax
from jax.experimental import pallas as pl
from jax.experimental.pallas import tpu as pltpu, tpu_sc as plsc

N = 10000
E = 160000
R = 8
D_IN = 256
G_DIM = 400
H = 100
HP = 128
HC = 256
TAG = 7

NC = 2
NS = 16
NW = NC * NS
EPT = 5120
E_PAD = NW * EPT
NB = EPT // 128
N_PAD = 10240
ROWS_PT = N_PAD // NS

_mesh = plsc.VectorSubcoreMesh(
    core_axis_name="c", subcore_axis_name="s", num_cores=NC, num_subcores=NS)


def _tc1_body(x_ref, wfc_ref, bfc_ref, w1_ref, a1_ref, h_ref, z_ref, st_ref):
    xb = x_ref[...]
    hb = jnp.maximum(jnp.dot(xb, wfc_ref[...],
                             preferred_element_type=jnp.float32) + bfc_ref[...], 0.0)
    h_ref[...] = hb
    st_parts = []
    for r in range(R):
        zr = jnp.dot(hb, w1_ref[r], preferred_element_type=jnp.float32)
        z_ref[r] = zr
        st_parts.append(jnp.dot(zr, a1_ref[r], preferred_element_type=jnp.float32))
    st_ref[...] = jnp.concatenate(st_parts, axis=1)


def _tc2_body(p_ref, w2_ref, a2_ref, z_ref, st_ref):
    g = jnp.maximum(p_ref[0] + p_ref[1], 0.0)
    st_parts = []
    for r in range(R):
        zr = jnp.dot(g, w2_ref[r], preferred_element_type=jnp.float32)
        z_ref[r] = zr
        st_parts.append(jnp.dot(zr, a2_ref[r], preferred_element_type=jnp.float32))
    st_ref[...] = jnp.concatenate(st_parts, axis=1)


def _tc3_body(h_ref, q_ref, wca_ref, wcb_ref, bc1_ref, wc2_ref, bc2_ref, o_ref):
    g2 = jnp.maximum(q_ref[0] + q_ref[1], 0.0)
    hidden = jnp.maximum(
        jnp.dot(h_ref[...], wca_ref[...], preferred_element_type=jnp.float32)
        + jnp.dot(g2, wcb_ref[...], preferred_element_type=jnp.float32)
        + bc1_ref[...], 0.0)
    o_ref[...] = jnp.dot(hidden, wc2_ref[...],
                         preferred_element_type=jnp.float32) + bc2_ref[...]


_BM = 400
_GRID = N // _BM


def _tc1(x, wfc, bfc, w1p, a1p):
    return pl.pallas_call(
        _tc1_body,
        grid=(_GRID,),
        in_specs=[
            pl.BlockSpec((_BM, D_IN), lambda i: (i, 0)),
            pl.BlockSpec((D_IN, G_DIM), lambda i: (0, 0)),
            pl.BlockSpec((1, G_DIM), lambda i: (0, 0)),
            pl.BlockSpec((R, G_DIM, HP), lambda i: (0, 0, 0)),
            pl.BlockSpec((R, HP, 2), lambda i: (0, 0, 0)),
        ],
        out_specs=[
            pl.BlockSpec((_BM, G_DIM), lambda i: (i, 0)),
            pl.BlockSpec((R, _BM, HP), lambda i: (0, i, 0)),
            pl.BlockSpec((_BM, 2 * R), lambda i: (i, 0)),
        ],
        out_shape=[
            jax.ShapeDtypeStruct((N, G_DIM), jnp.float32),
            jax.ShapeDtypeStruct((R, N, HP), jnp.float32),
            jax.ShapeDtypeStruct((N, 2 * R), jnp.float32),
        ],
    )(x, wfc, bfc, w1p, a1p)


def _tc2(p, w2p, a2p):
    return pl.pallas_call(
        _tc2_body,
        grid=(_GRID,),
        in_specs=[
            pl.BlockSpec((2, _BM, HP), lambda i: (0, i, 0)),
            pl.BlockSpec((R, HP, HP), lambda i: (0, 0, 0)),
            pl.BlockSpec((R, HP, 2), lambda i: (0, 0, 0)),
        ],
        out_specs=[
            pl.BlockSpec((R, _BM, HP), lambda i: (0, i, 0)),
            pl.BlockSpec((_BM, 2 * R), lambda i: (i, 0)),
        ],
        out_shape=[
            jax.ShapeDtypeStruct((R, N, HP), jnp.float32),
            jax.ShapeDtypeStruct((N, 2 * R), jnp.float32),
        ],
    )(p, w2p, a2p)


def _tc3(h, q, wca, wcb, bc1, wc2p, bc2p):
    return pl.pallas_call(
        _tc3_body,
        grid=(_GRID,),
        in_specs=[
            pl.BlockSpec((_BM, G_DIM), lambda i: (i, 0)),
            pl.BlockSpec((2, _BM, HP), lambda i: (0, i, 0)),
            pl.BlockSpec((G_DIM, HC), lambda i: (0, 0)),
            pl.BlockSpec((HP, HC), lambda i: (0, 0)),
            pl.BlockSpec((1, HC), lambda i: (0, 0)),
            pl.BlockSpec((HC, 8), lambda i: (0, 0)),
            pl.BlockSpec((1, 8), lambda i: (0, 0)),
        ],
        out_specs=[pl.BlockSpec((_BM, 8), lambda i: (i, 0))],
        out_shape=[jax.ShapeDtypeStruct((N, 8), jnp.float32)],
    )(h, q, wca, wcb, bc1, wc2p, bc2p)


def _zero16():
    return jnp.zeros((16,), jnp.float32)


_GDN = lax.GatherDimensionNumbers(
    offset_dims=(), collapsed_slice_dims=(0,), start_index_map=(0,))


def _bcast_lane(vec, i):
    idx = jnp.full((16, 1), i, jnp.int32)
    return lax.gather(vec, idx, dimension_numbers=_GDN, slice_sizes=(1,),
                      mode=lax.GatherScatterMode.PROMISE_IN_BOUNDS)


def _sca_body(src_hbm, dst_hbm, et_hbm, st_hbm,
              ex_hbm, idx1_hbm, den_hbm,
              srcv, dstv, etv, idx1v, exv, idxs2, idxt2, dst2,
              sbuf, tbuf, zbuf, den_sp, sg1, sg2):
    cid = lax.axis_index("c")
    sid = lax.axis_index("s")
    wid = cid * NS + sid
    base = wid * EPT

    def zb(k, _):
        zbuf[pl.ds(k * 16, 16)] = _zero16()
        return 0
    lax.fori_loop(0, ROWS_PT // 16, zb, 0)
    pltpu.sync_copy(zbuf, den_sp.at[pl.ds(sid * ROWS_PT, ROWS_PT)])

    pltpu.sync_copy(src_hbm.at[pl.ds(base, EPT)], srcv)
    pltpu.sync_copy(dst_hbm.at[pl.ds(base, EPT)], dstv)
    pltpu.sync_copy(et_hbm.at[pl.ds(base, EPT)], etv)

    def idx_batch(b, _):
        for g in range(8):
            off = b * 128 + g * 16
            s_i = srcv[pl.ds(off, 16)]
            d_i = dstv[pl.ds(off, 16)]
            t_i = etv[pl.ds(off, 16)]
            idxs2[b, pl.ds(g * 16, 16)] = s_i * 16 + t_i * 2
            idxt2[b, pl.ds(g * 16, 16)] = d_i * 16 + t_i * 2 + 1
            dst2[b, pl.ds(g * 16, 16)] = d_i
            idx1v[pl.ds(off, 16)] = t_i * N + s_i
        return 0
    lax.fori_loop(0, NB, idx_batch, 0)

    plsc.subcore_barrier()

    def ex_batch(b, _):
        cp1 = pltpu.async_copy(st_hbm.at[idxs2.at[b]], sbuf, sg1)
        cp2 = pltpu.async_copy(st_hbm.at[idxt2.at[b]], tbuf, sg2)
        cp1.wait()
        cp2.wait()
        for g in range(8):
            off = b * 128 + g * 16
            sc = sbuf[pl.ds(g * 16, 16)] + tbuf[pl.ds(g * 16, 16)]
            sc = jnp.where(sc >= 0.0, sc, sc * 0.2)
            gid = base + off + lax.iota(jnp.int32, 16)
            ex = jnp.where(gid < E, jnp.exp(sc), 0.0)
            exv[pl.ds(off, 16)] = ex
        pltpu.sync_copy(exv.at[pl.ds(b * 128, 128)],
                        den_sp.at[dst2.at[b]], add=True)
        return 0
    lax.fori_loop(0, NB, ex_batch, 0)

    pltpu.sync_copy(exv, ex_hbm.at[pl.ds(base, EPT)])
    pltpu.sync_copy(idx1v, idx1_hbm.at[pl.ds(base, EPT)])

    plsc.subcore_barrier()
    pltpu.sync_copy(den_sp.at[pl.ds(sid * ROWS_PT, ROWS_PT)],
                    den_hbm.at[pl.ds(cid * N_PAD + sid * ROWS_PT, ROWS_PT)])


def _sca(src, dst, et, st_flat):
    f = pl.kernel(
        _sca_body,
        out_type=(
            jax.ShapeDtypeStruct((E_PAD,), jnp.float32),
            jax.ShapeDtypeStruct((E_PAD,), jnp.int32),
            jax.ShapeDtypeStruct((2 * N_PAD,), jnp.float32),
        ),
        mesh=_mesh,
        scratch_types=[
            pltpu.VMEM((EPT,), jnp.int32),
            pltpu.VMEM((EPT,), jnp.int32),
            pltpu.VMEM((EPT,), jnp.int32),
            pltpu.VMEM((EPT,), jnp.int32),
            pltpu.VMEM((EPT,), jnp.float32),
            pltpu.VMEM((NB, 128), jnp.int32),
            pltpu.VMEM((NB, 128), jnp.int32),
            pltpu.VMEM((NB, 128), jnp.int32),
            pltpu.VMEM((128,), jnp.float32),
            pltpu.VMEM((128,), jnp.float32),
            pltpu.VMEM((ROWS_PT,), jnp.float32),
            pltpu.VMEM_SHARED((N_PAD,), jnp.float32),
            pltpu.SemaphoreType.DMA,
            pltpu.SemaphoreType.DMA,
        ],
    )
    return f(src, dst, et, st_flat)


def _scb_body(idx1_hbm, dst_hbm, ex_hbm, den_hbm, z_hbm,
              outp_hbm,
              idx12, dst2, d12, exv, rowbuf, p0buf, p1buf, albuf,
              out_sp, srow, sg1, sg2):
    cid = lax.axis_index("c")
    sid = lax.axis_index("s")
    wid = cid * NS + sid
    base = wid * EPT

    def zr(r, _):
        for c in range(HP // 16):
            rowbuf[r, pl.ds(c * 16, 16)] = _zero16()
        return 0
    lax.fori_loop(0, 128, zr, 0)
    for j in range(ROWS_PT // 128):
        pltpu.sync_copy(rowbuf, out_sp.at[pl.ds(sid * ROWS_PT + j * 128, 128)])

    def ld(b, _):
        pltpu.sync_copy(idx1_hbm.at[pl.ds(base + b * 128, 128)], idx12.at[b])
        pltpu.sync_copy(dst_hbm.at[pl.ds(base + b * 128, 128)], dst2.at[b])
        return 0
    lax.fori_loop(0, NB, ld, 0)
    pltpu.sync_copy(ex_hbm.at[pl.ds(base, EPT)], exv)

    def mk_d1(b, _):
        for g in range(8):
            d12[b, pl.ds(g * 16, 16)] = dst2[b, pl.ds(g * 16, 16)] + N_PAD
        return 0
    lax.fori_loop(0, NB, mk_d1, 0)

    plsc.subcore_barrier()

    def agg_batch(b, _):
        row_cp = pltpu.async_copy(z_hbm.at[idx12.at[b]], rowbuf, srow)
        cp0 = pltpu.async_copy(den_hbm.at[dst2.at[b]], p0buf, sg1)
        cp1 = pltpu.async_copy(den_hbm.at[d12.at[b]], p1buf, sg2)
        cp0.wait()
        cp1.wait()

        def alpha_group(g, _):
            off = b * 128 + g * 16
            ex = exv[pl.ds(off, 16)]
            den = p0buf[pl.ds(g * 16, 16)] + p1buf[pl.ds(g * 16, 16)] + 1e-16
            albuf[pl.ds(g * 16, 16)] = ex / den
            return 0
        lax.fori_loop(0, 8, alpha_group, 0)
        row_cp.wait()

        def scale_group(g, _):
            al = albuf[pl.ds(g * 16, 16)]
            for i in range(16):
                bc = _bcast_lane(al, i)
                row = g * 16 + i
                for c in range(HP // 16):
                    rowbuf[row, pl.ds(c * 16, 16)] = (
                        rowbuf[row, pl.ds(c * 16, 16)] * bc)
            return 0
        lax.fori_loop(0, 8, scale_group, 0)

        pltpu.sync_copy(rowbuf, out_sp.at[dst2.at[b]], add=True)
        return 0
    lax.fori_loop(0, NB, agg_batch, 0)

    plsc.subcore_barrier()
    pltpu.sync_copy(out_sp.at[pl.ds(sid * ROWS_PT, ROWS_PT)],
                    outp_hbm.at[cid, pl.ds(sid * ROWS_PT, ROWS_PT)])


def _scb(idx1, dst, ex, den, z_flat):
    f = pl.kernel(
        _scb_body,
        out_type=jax.ShapeDtypeStruct((2, N_PAD, HP), jnp.float32),
        mesh=_mesh,
        scratch_types=[
            pltpu.VMEM((NB, 128), jnp.int32),
            pltpu.VMEM((NB, 128), jnp.int32),
            pltpu.VMEM((NB, 128), jnp.int32),
            pltpu.VMEM((EPT,), jnp.float32),
            pltpu.VMEM((128, HP), jnp.float32),
            pltpu.VMEM((128,), jnp.float32),
            pltpu.VMEM((128,), jnp.float32),
            pltpu.VMEM((128,), jnp.float32),
            pltpu.VMEM_SHARED((N_PAD, HP), jnp.float32),
            pltpu.SemaphoreType.DMA,
            pltpu.SemaphoreType.DMA,
            pltpu.SemaphoreType.DMA,
        ],
    )
    return f(idx1, dst, ex, den, z_flat)


def kernel(x, edge_index, edge_type, W_fc, b_fc, W1, a1_src, a1_dst,
           W2, a2_src, a2_dst, Wc1, bc1, Wc2, bc2):
    f32 = jnp.float32
    src = jnp.pad(edge_index[0].astype(jnp.int32), (0, E_PAD - E))
    dst = jnp.pad(edge_index[1].astype(jnp.int32), (0, E_PAD - E))
    et = jnp.pad(edge_type.astype(jnp.int32), (0, E_PAD - E))

    w1p = jnp.pad(W1.astype(f32), ((0, 0), (0, 0), (0, HP - H)))
    a1p = jnp.pad(jnp.stack([a1_src, a1_dst], axis=-1).astype(f32),
                  ((0, HP - H), (0, 0)))
    a1p = jnp.broadcast_to(a1p[None], (R, HP, 2))
    w2p = jnp.pad(W2.astype(f32), ((0, 0), (0, HP - H), (0, HP - H)))
    a2p = jnp.pad(jnp.stack([a2_src, a2_dst], axis=-1).astype(f32),
                  ((0, HP - H), (0, 0)))
    a2p = jnp.broadcast_to(a2p[None], (R, HP, 2))
    wca = Wc1[:G_DIM].astype(f32)
    wcb = jnp.pad(Wc1[G_DIM:].astype(f32), ((0, HP - H), (0, 0)))
    wc2p = jnp.pad(Wc2.astype(f32), ((0, 0), (0, 8 - TAG)))
    bc2p = jnp.pad(bc2.astype(f32), (0, 8 - TAG)).reshape(1, 8)

    h, z1, st1 = _tc1(x.astype(f32), W_fc.astype(f32),
                      b_fc.astype(f32).reshape(1, G_DIM), w1p, a1p)
    ex1, idx1a, den1 = _sca(src, dst, et, st1.reshape(-1))
    p1 = _scb(idx1a, dst, ex1, den1, z1.reshape(R * N, HP))

    z2, st2 = _tc2(p1, w2p, a2p)
    ex2, idx2a, den2 = _sca(src, dst, et, st2.reshape(-1))
    p2 = _scb(idx2a, dst, ex2, den2, z2.reshape(R * N, HP))

    (out8,) = _tc3(h, p2, wca, wcb, bc1.astype(f32).reshape(1, HC), wc2p, bc2p)
    return out8[:, :TAG]


# speedup vs baseline: 14.3915x; 1.1836x over previous
import functools

import jax
import jax.numpy as jnp
from jax import lax
from jax.experimental import pallas as pl
from jax.experimental.pallas import tpu as pltpu, tpu_sc as plsc

N = 10000
E = 160000
R = 8
D_IN = 256
G_DIM = 400
H = 100
HP = 128
HC = 256
TAG = 7

NC = 2
NS = 16
NW = NC * NS
EPT = 5120
E_PAD = NW * EPT
NB = EPT // 128
N_PAD = 10240
ROWS_PT = N_PAD // NS
NOUT = N
OCHUNK = 632

_mesh = plsc.VectorSubcoreMesh(
    core_axis_name="c", subcore_axis_name="s", num_cores=NC, num_subcores=NS)


_BF = jnp.bfloat16


def _tc1_body(x_ref, wfc_ref, bfc_ref, w1_ref, a1_ref, h_ref, z_ref, st_ref):
    xb = x_ref[...]
    hb = jnp.maximum(jnp.dot(xb, wfc_ref[...],
                             preferred_element_type=jnp.float32) + bfc_ref[...], 0.0)
    h_ref[...] = hb
    st_parts = []
    for r in range(R):
        zr = jnp.dot(hb, w1_ref[r], preferred_element_type=jnp.float32)
        z_ref[r] = zr
        st_parts.append(jnp.dot(zr, a1_ref[r], preferred_element_type=jnp.float32))
    st_ref[...] = jnp.concatenate(st_parts, axis=1)


def _tc2_body(p_ref, w2_ref, a2_ref, z_ref, st_ref):
    g = jnp.maximum(p_ref[0] + p_ref[1], 0.0)
    st_parts = []
    for r in range(R):
        zr = jnp.dot(g, w2_ref[r], preferred_element_type=jnp.float32)
        z_ref[r] = zr
        st_parts.append(jnp.dot(zr, a2_ref[r], preferred_element_type=jnp.float32))
    st_ref[...] = jnp.concatenate(st_parts, axis=1)


def _tc3_body(h_ref, q_ref, wca_ref, wcb_ref, bc1_ref, wc2_ref, bc2_ref, o_ref):
    g2 = jnp.maximum(q_ref[0] + q_ref[1], 0.0)
    hidden = jnp.maximum(
        jnp.dot(h_ref[...], wca_ref[...], preferred_element_type=jnp.float32)
        + jnp.dot(g2, wcb_ref[...], preferred_element_type=jnp.float32)
        + bc1_ref[...], 0.0)
    o_ref[...] = jnp.dot(hidden, wc2_ref[...],
                         preferred_element_type=jnp.float32) + bc2_ref[...]


_BM = 400
_GRID = N // _BM


def _tc1(x, wfc, bfc, w1p, a1p):
    return pl.pallas_call(
        _tc1_body,
        grid=(_GRID,),
        in_specs=[
            pl.BlockSpec((_BM, D_IN), lambda i: (i, 0)),
            pl.BlockSpec((D_IN, G_DIM), lambda i: (0, 0)),
            pl.BlockSpec((1, G_DIM), lambda i: (0, 0)),
            pl.BlockSpec((R, G_DIM, HP), lambda i: (0, 0, 0)),
            pl.BlockSpec((R, HP, 2), lambda i: (0, 0, 0)),
        ],
        out_specs=[
            pl.BlockSpec((_BM, G_DIM), lambda i: (i, 0)),
            pl.BlockSpec((R, _BM, HP), lambda i: (0, i, 0)),
            pl.BlockSpec((_BM, 2 * R), lambda i: (i, 0)),
        ],
        out_shape=[
            jax.ShapeDtypeStruct((N, G_DIM), jnp.float32),
            jax.ShapeDtypeStruct((R, N, HP), jnp.float32),
            jax.ShapeDtypeStruct((N, 2 * R), jnp.float32),
        ],
    )(x, wfc, bfc, w1p, a1p)


def _tc2(p, w2p, a2p):
    return pl.pallas_call(
        _tc2_body,
        grid=(_GRID,),
        in_specs=[
            pl.BlockSpec((2, _BM, HP), lambda i: (0, i, 0)),
            pl.BlockSpec((R, HP, HP), lambda i: (0, 0, 0)),
            pl.BlockSpec((R, HP, 2), lambda i: (0, 0, 0)),
        ],
        out_specs=[
            pl.BlockSpec((R, _BM, HP), lambda i: (0, i, 0)),
            pl.BlockSpec((_BM, 2 * R), lambda i: (i, 0)),
        ],
        out_shape=[
            jax.ShapeDtypeStruct((R, N, HP), jnp.float32),
            jax.ShapeDtypeStruct((N, 2 * R), jnp.float32),
        ],
    )(p, w2p, a2p)


def _tc3(h, q, wca, wcb, bc1, wc2p, bc2p):
    return pl.pallas_call(
        _tc3_body,
        grid=(_GRID,),
        in_specs=[
            pl.BlockSpec((_BM, G_DIM), lambda i: (i, 0)),
            pl.BlockSpec((2, _BM, HP), lambda i: (0, i, 0)),
            pl.BlockSpec((G_DIM, HC), lambda i: (0, 0)),
            pl.BlockSpec((HP, HC), lambda i: (0, 0)),
            pl.BlockSpec((1, HC), lambda i: (0, 0)),
            pl.BlockSpec((HC, 8), lambda i: (0, 0)),
            pl.BlockSpec((1, 8), lambda i: (0, 0)),
        ],
        out_specs=[pl.BlockSpec((_BM, 8), lambda i: (i, 0))],
        out_shape=[jax.ShapeDtypeStruct((N, 8), jnp.float32)],
    )(h, q, wca, wcb, bc1, wc2p, bc2p)


def _zero16():
    return jnp.zeros((16,), jnp.float32)


_GDN = lax.GatherDimensionNumbers(
    offset_dims=(), collapsed_slice_dims=(0,), start_index_map=(0,))


def _bcast_lane(vec, i):
    idx = jnp.full((16, 1), i, jnp.int32)
    return lax.gather(vec, idx, dimension_numbers=_GDN, slice_sizes=(1,),
                      mode=lax.GatherScatterMode.PROMISE_IN_BOUNDS)


def _sca_body(src_hbm, dst_hbm, et_hbm, st_hbm,
              ex_hbm, idx1_hbm, den_hbm,
              srcv, dstv, etv, idx1v, exv, idxs2, idxt2, dst2,
              sbuf, tbuf, sbuf2, tbuf2, zbuf, den_sp, sg1, sg2, sg3, sg4):
    cid = lax.axis_index("c")
    sid = lax.axis_index("s")
    wid = cid * NS + sid
    base = wid * EPT

    def zb(k, _):
        zbuf[pl.ds(k * 16, 16)] = _zero16()
        return 0
    lax.fori_loop(0, ROWS_PT // 16, zb, 0)
    pltpu.sync_copy(zbuf, den_sp.at[pl.ds(sid * ROWS_PT, ROWS_PT)])

    pltpu.sync_copy(src_hbm.at[pl.ds(base, EPT)], srcv)
    pltpu.sync_copy(dst_hbm.at[pl.ds(base, EPT)], dstv)
    pltpu.sync_copy(et_hbm.at[pl.ds(base, EPT)], etv)

    def idx_batch(b, _):
        for g in range(8):
            off = b * 128 + g * 16
            s_i = srcv[pl.ds(off, 16)]
            d_i = dstv[pl.ds(off, 16)]
            t_i = etv[pl.ds(off, 16)]
            idxs2[b, pl.ds(g * 16, 16)] = s_i * 16 + t_i * 2
            idxt2[b, pl.ds(g * 16, 16)] = d_i * 16 + t_i * 2 + 1
            dst2[b, pl.ds(g * 16, 16)] = d_i
            idx1v[pl.ds(off, 16)] = t_i * N + s_i
        return 0
    lax.fori_loop(0, NB, idx_batch, 0)

    plsc.subcore_barrier()

    def fire(b, sb, tb, g1, g2):
        pltpu.async_copy(st_hbm.at[idxs2.at[b]], sb, g1)
        pltpu.async_copy(st_hbm.at[idxt2.at[b]], tb, g2)

    def drain(sb, tb, g1, g2):
        pltpu.make_async_copy(st_hbm.at[idxs2.at[0]], sb, g1).wait()
        pltpu.make_async_copy(st_hbm.at[idxt2.at[0]], tb, g2).wait()

    def ex_batch(b, sb, tb):
        for g in range(8):
            off = b * 128 + g * 16
            sc = sb[pl.ds(g * 16, 16)] + tb[pl.ds(g * 16, 16)]
            sc = jnp.where(sc >= 0.0, sc, sc * 0.2)
            gid = base + off + lax.iota(jnp.int32, 16)
            ex = jnp.where(gid < E, jnp.exp(sc), 0.0)
            exv[pl.ds(off, 16)] = ex
        pltpu.sync_copy(exv.at[pl.ds(b * 128, 128)],
                        den_sp.at[dst2.at[b]], add=True)

    fire(0, sbuf, tbuf, sg1, sg2)

    def pair(k, _):
        b0 = 2 * k
        fire(b0 + 1, sbuf2, tbuf2, sg3, sg4)
        drain(sbuf, tbuf, sg1, sg2)
        ex_batch(b0, sbuf, tbuf)

        @pl.when(k < NB // 2 - 1)
        def _():
            fire(b0 + 2, sbuf, tbuf, sg1, sg2)
        drain(sbuf2, tbuf2, sg3, sg4)
        ex_batch(b0 + 1, sbuf2, tbuf2)
        return 0
    lax.fori_loop(0, NB // 2, pair, 0)

    pltpu.sync_copy(exv, ex_hbm.at[pl.ds(base, EPT)])
    pltpu.sync_copy(idx1v, idx1_hbm.at[pl.ds(base, EPT)])

    plsc.subcore_barrier()
    pltpu.sync_copy(den_sp.at[pl.ds(sid * ROWS_PT, ROWS_PT)],
                    den_hbm.at[pl.ds(cid * N_PAD + sid * ROWS_PT, ROWS_PT)])


def _sca(src, dst, et, st_flat):
    f = pl.kernel(
        _sca_body,
        out_type=(
            jax.ShapeDtypeStruct((E_PAD,), jnp.float32),
            jax.ShapeDtypeStruct((E_PAD,), jnp.int32),
            jax.ShapeDtypeStruct((2 * N_PAD,), jnp.float32),
        ),
        mesh=_mesh,
        scratch_types=[
            pltpu.VMEM((EPT,), jnp.int32),
            pltpu.VMEM((EPT,), jnp.int32),
            pltpu.VMEM((EPT,), jnp.int32),
            pltpu.VMEM((EPT,), jnp.int32),
            pltpu.VMEM((EPT,), jnp.float32),
            pltpu.VMEM((NB, 128), jnp.int32),
            pltpu.VMEM((NB, 128), jnp.int32),
            pltpu.VMEM((NB, 128), jnp.int32),
            pltpu.VMEM((128,), jnp.float32),
            pltpu.VMEM((128,), jnp.float32),
            pltpu.VMEM((128,), jnp.float32),
            pltpu.VMEM((128,), jnp.float32),
            pltpu.VMEM((ROWS_PT,), jnp.float32),
            pltpu.VMEM_SHARED((N_PAD,), jnp.float32),
            pltpu.SemaphoreType.DMA,
            pltpu.SemaphoreType.DMA,
            pltpu.SemaphoreType.DMA,
            pltpu.SemaphoreType.DMA,
        ],
    )
    return f(src, dst, et, st_flat)


RB = 128
NB2 = EPT // RB
SCALE_CHUNKS = 7


def _scb_body(idx1_hbm, dst_hbm, ex_hbm, den_hbm, z_hbm,
              outp_hbm,
              idx12, dst2, dbufA, dbufB, ebufA, ebufB, rbufA, rbufB,
              out_sp, den_sp2, semA, semB, semDA, semDB, semEA, semEB,
              semOA, semOB):
    cid = lax.axis_index("c")
    sid = lax.axis_index("s")
    wid = cid * NS + sid
    base = wid * EPT

    def zr(r, _):
        for c in range(HP // 16):
            rbufA[r, pl.ds(c * 16, 16)] = _zero16()
        return 0
    lax.fori_loop(0, RB, zr, 0)
    obase = pl.multiple_of(jnp.minimum(sid * OCHUNK, NOUT - OCHUNK), 8)

    def zcp(j, _):
        pltpu.sync_copy(rbufA, out_sp.at[pl.ds(
            pl.multiple_of(obase + j * RB, 8), RB)])
        return 0
    lax.fori_loop(0, OCHUNK // RB, zcp, 0)
    pltpu.sync_copy(rbufA.at[pl.ds(0, OCHUNK % RB)],
                    out_sp.at[pl.ds(obase + (OCHUNK // RB) * RB,
                                    OCHUNK % RB)])

    def dj(j, _):
        o = sid * ROWS_PT + j * RB
        pltpu.sync_copy(den_hbm.at[pl.ds(o, RB)], dbufA)
        pltpu.sync_copy(den_hbm.at[pl.ds(N_PAD + o, RB)], ebufA)
        for q in range(RB // 16):
            dbufA[pl.ds(q * 16, 16)] = (dbufA[pl.ds(q * 16, 16)]
                                        + ebufA[pl.ds(q * 16, 16)])
        pltpu.sync_copy(dbufA, den_sp2.at[pl.ds(o, RB)])
        return 0
    lax.fori_loop(0, ROWS_PT // RB, dj, 0)

    def ld(b, _):
        pltpu.sync_copy(idx1_hbm.at[pl.ds(base + b * RB, RB)], idx12.at[b])
        pltpu.sync_copy(dst_hbm.at[pl.ds(base + b * RB, RB)], dst2.at[b])
        return 0
    lax.fori_loop(0, NB2, ld, 0)

    plsc.subcore_barrier()

    def process(b, rbuf, dbuf, ebuf, so):
        def scale_group(g, _):
            den16 = dbuf[pl.ds(g * 16, 16)]
            al = ebuf[pl.ds(g * 16, 16)] / (den16 + 1e-16)
            for i in range(16):
                bc = _bcast_lane(al, i)
                row = g * 16 + i
                for c in range(SCALE_CHUNKS):
                    rbuf[row, pl.ds(c * 16, 16)] = (
                        rbuf[row, pl.ds(c * 16, 16)] * bc)
            return 0
        lax.fori_loop(0, RB // 16, scale_group, 0)
        pltpu.async_copy(rbuf, out_sp.at[dst2.at[b]], so, add=True)

    def fire(b, rbuf, dbuf, ebuf, sr, sd, se):
        pltpu.async_copy(z_hbm.at[idx12.at[b]], rbuf, sr)
        pltpu.async_copy(den_sp2.at[dst2.at[b]], dbuf, sd)
        pltpu.async_copy(ex_hbm.at[pl.ds(base + b * RB, RB)], ebuf, se)

    def drain(rbuf, dbuf, ebuf, sr, sd, se):
        pltpu.make_async_copy(z_hbm.at[idx12.at[0]], rbuf, sr).wait()
        pltpu.make_async_copy(den_sp2.at[dst2.at[0]], dbuf, sd).wait()
        pltpu.make_async_copy(ex_hbm.at[pl.ds(base, RB)], ebuf, se).wait()

    def drain_sc(rbuf, so):
        pltpu.make_async_copy(rbuf, out_sp.at[dst2.at[0]], so).wait()

    fire(0, rbufA, dbufA, ebufA, semA, semDA, semEA)

    def pair(k, _):
        b0 = 2 * k
        fire(b0 + 1, rbufB, dbufB, ebufB, semB, semDB, semEB)
        drain(rbufA, dbufA, ebufA, semA, semDA, semEA)
        process(b0, rbufA, dbufA, ebufA, semOA)

        @pl.when(k < NB2 // 2 - 1)
        def _():
            drain_sc(rbufA, semOA)
            fire(b0 + 2, rbufA, dbufA, ebufA, semA, semDA, semEA)
        drain(rbufB, dbufB, ebufB, semB, semDB, semEB)

        @pl.when(k > 0)
        def _():
            drain_sc(rbufB, semOB)
        process(b0 + 1, rbufB, dbufB, ebufB, semOB)
        return 0
    lax.fori_loop(0, NB2 // 2, pair, 0)
    drain_sc(rbufA, semOA)
    drain_sc(rbufB, semOB)

    plsc.subcore_barrier()
    pltpu.sync_copy(out_sp.at[pl.ds(obase, OCHUNK)],
                    outp_hbm.at[cid, pl.ds(obase, OCHUNK)])


def _scb(idx1, dst, ex, den, z_flat):
    f = pl.kernel(
        _scb_body,
        out_type=jax.ShapeDtypeStruct((2, NOUT, HP), jnp.float32),
        mesh=_mesh,
        scratch_types=[
            pltpu.VMEM((NB2, RB), jnp.int32),
            pltpu.VMEM((NB2, RB), jnp.int32),
            pltpu.VMEM((RB,), jnp.float32),
            pltpu.VMEM((RB,), jnp.float32),
            pltpu.VMEM((RB,), jnp.float32),
            pltpu.VMEM((RB,), jnp.float32),
            pltpu.VMEM((RB, HP), jnp.float32),
            pltpu.VMEM((RB, HP), jnp.float32),
            pltpu.VMEM_SHARED((NOUT, HP), jnp.float32),
            pltpu.VMEM_SHARED((N_PAD,), jnp.float32),
            pltpu.SemaphoreType.DMA,
            pltpu.SemaphoreType.DMA,
            pltpu.SemaphoreType.DMA,
            pltpu.SemaphoreType.DMA,
            pltpu.SemaphoreType.DMA,
            pltpu.SemaphoreType.DMA,
            pltpu.SemaphoreType.DMA,
            pltpu.SemaphoreType.DMA,
        ],
    )
    return f(idx1, dst, ex, den, z_flat)


def kernel(x, edge_index, edge_type, W_fc, b_fc, W1, a1_src, a1_dst,
           W2, a2_src, a2_dst, Wc1, bc1, Wc2, bc2):
    f32 = jnp.float32
    src = jnp.pad(edge_index[0].astype(jnp.int32), (0, E_PAD - E))
    dst = jnp.pad(edge_index[1].astype(jnp.int32), (0, E_PAD - E))
    et = jnp.pad(edge_type.astype(jnp.int32), (0, E_PAD - E))

    w1p = jnp.pad(W1.astype(f32), ((0, 0), (0, 0), (0, HP - H)))
    a1p = jnp.pad(jnp.stack([a1_src, a1_dst], axis=-1).astype(f32),
                  ((0, HP - H), (0, 0)))
    a1p = jnp.broadcast_to(a1p[None], (R, HP, 2))
    w2p = jnp.pad(W2.astype(f32), ((0, 0), (0, HP - H), (0, HP - H)))
    a2p = jnp.pad(jnp.stack([a2_src, a2_dst], axis=-1).astype(f32),
                  ((0, HP - H), (0, 0)))
    a2p = jnp.broadcast_to(a2p[None], (R, HP, 2))
    wca = Wc1[:G_DIM].astype(f32)
    wcb = jnp.pad(Wc1[G_DIM:].astype(f32), ((0, HP - H), (0, 0)))
    wc2p = jnp.pad(Wc2.astype(f32), ((0, 0), (0, 8 - TAG)))
    bc2p = jnp.pad(bc2.astype(f32), (0, 8 - TAG)).reshape(1, 8)

    h, z1, st1 = _tc1(x.astype(f32), W_fc.astype(f32),
                      b_fc.astype(f32).reshape(1, G_DIM), w1p, a1p)
    ex1, idx1a, den1 = _sca(src, dst, et, st1.reshape(-1))
    p1 = _scb(idx1a, dst, ex1, den1, z1.reshape(R * N, HP))

    z2, st2 = _tc2(p1, w2p, a2p)
    ex2, idx2a, den2 = _sca(src, dst, et, st2.reshape(-1))
    p2 = _scb(idx2a, dst, ex2, den2, z2.reshape(R * N, HP))

    (out8,) = _tc3(h, p2, wca, wcb, bc1.astype(f32).reshape(1, HC), wc2p, bc2p)
    return out8[:, :TAG]


# speedup vs baseline: 14.4169x; 1.0018x over previous
import jax
import jax.numpy as jnp
from jax import lax
from jax.experimental import pallas as pl
from jax.experimental.pallas import tpu as pltpu, tpu_sc as plsc

N = 10000
E = 160000
R = 8
D_IN = 256
G_DIM = 400
H = 100
HP = 128
HC = 256
TAG = 7

NC = 2
NS = 16
NW = NC * NS
EPT = 5120
E_PAD = NW * EPT
NB = EPT // 128
N_PAD = 10240
ROWS_PT = N_PAD // NS
NOUT = N
OCHUNK = 632

_mesh = plsc.VectorSubcoreMesh(
    core_axis_name="c", subcore_axis_name="s", num_cores=NC, num_subcores=NS)


def _tc1_body(x_ref, wfc_ref, bfc_ref, w1_ref, a1_ref, h_ref, z_ref, st_ref):
    xb = x_ref[...]
    hb = jnp.maximum(jnp.dot(xb, wfc_ref[...],
                             preferred_element_type=jnp.float32) + bfc_ref[...], 0.0)
    h_ref[...] = hb
    st_parts = []
    for r in range(R):
        zr = jnp.dot(hb, w1_ref[r], preferred_element_type=jnp.float32)
        z_ref[r] = zr
        st_parts.append(jnp.dot(zr, a1_ref[r], preferred_element_type=jnp.float32))
    st_ref[...] = jnp.concatenate(st_parts, axis=1)


def _tc2_body(p_ref, w2_ref, a2_ref, z_ref, st_ref):
    g = jnp.maximum(p_ref[0] + p_ref[1], 0.0)
    st_parts = []
    for r in range(R):
        zr = jnp.dot(g, w2_ref[r], preferred_element_type=jnp.float32)
        z_ref[r] = zr
        st_parts.append(jnp.dot(zr, a2_ref[r], preferred_element_type=jnp.float32))
    st_ref[...] = jnp.concatenate(st_parts, axis=1)


def _tc3_body(h_ref, q_ref, wca_ref, wcb_ref, bc1_ref, wc2_ref, bc2_ref, o_ref):
    g2 = jnp.maximum(q_ref[0] + q_ref[1], 0.0)
    hidden = jnp.maximum(
        jnp.dot(h_ref[...], wca_ref[...], preferred_element_type=jnp.float32)
        + jnp.dot(g2, wcb_ref[...], preferred_element_type=jnp.float32)
        + bc1_ref[...], 0.0)
    o_ref[...] = jnp.dot(hidden, wc2_ref[...],
                         preferred_element_type=jnp.float32) + bc2_ref[...]


_BM = 400
_GRID = N // _BM


def _tc1(x, wfc, bfc, w1p, a1p):
    return pl.pallas_call(
        _tc1_body,
        grid=(_GRID,),
        in_specs=[
            pl.BlockSpec((_BM, D_IN), lambda i: (i, 0)),
            pl.BlockSpec((D_IN, G_DIM), lambda i: (0, 0)),
            pl.BlockSpec((1, G_DIM), lambda i: (0, 0)),
            pl.BlockSpec((R, G_DIM, HP), lambda i: (0, 0, 0)),
            pl.BlockSpec((R, HP, 2), lambda i: (0, 0, 0)),
        ],
        out_specs=[
            pl.BlockSpec((_BM, G_DIM), lambda i: (i, 0)),
            pl.BlockSpec((R, _BM, HP), lambda i: (0, i, 0)),
            pl.BlockSpec((_BM, 2 * R), lambda i: (i, 0)),
        ],
        out_shape=[
            jax.ShapeDtypeStruct((N, G_DIM), jnp.float32),
            jax.ShapeDtypeStruct((R, N, HP), jnp.float32),
            jax.ShapeDtypeStruct((N, 2 * R), jnp.float32),
        ],
    )(x, wfc, bfc, w1p, a1p)


def _tc2(p, w2p, a2p):
    return pl.pallas_call(
        _tc2_body,
        grid=(_GRID,),
        in_specs=[
            pl.BlockSpec((2, _BM, HP), lambda i: (0, i, 0)),
            pl.BlockSpec((R, HP, HP), lambda i: (0, 0, 0)),
            pl.BlockSpec((R, HP, 2), lambda i: (0, 0, 0)),
        ],
        out_specs=[
            pl.BlockSpec((R, _BM, HP), lambda i: (0, i, 0)),
            pl.BlockSpec((_BM, 2 * R), lambda i: (i, 0)),
        ],
        out_shape=[
            jax.ShapeDtypeStruct((R, N, HP), jnp.float32),
            jax.ShapeDtypeStruct((N, 2 * R), jnp.float32),
        ],
    )(p, w2p, a2p)


def _tc3(h, q, wca, wcb, bc1, wc2p, bc2p):
    return pl.pallas_call(
        _tc3_body,
        grid=(_GRID,),
        in_specs=[
            pl.BlockSpec((_BM, G_DIM), lambda i: (i, 0)),
            pl.BlockSpec((2, _BM, HP), lambda i: (0, i, 0)),
            pl.BlockSpec((G_DIM, HC), lambda i: (0, 0)),
            pl.BlockSpec((HP, HC), lambda i: (0, 0)),
            pl.BlockSpec((1, HC), lambda i: (0, 0)),
            pl.BlockSpec((HC, 8), lambda i: (0, 0)),
            pl.BlockSpec((1, 8), lambda i: (0, 0)),
        ],
        out_specs=[pl.BlockSpec((_BM, 8), lambda i: (i, 0))],
        out_shape=[jax.ShapeDtypeStruct((N, 8), jnp.float32)],
    )(h, q, wca, wcb, bc1, wc2p, bc2p)


def _zero16():
    return jnp.zeros((16,), jnp.float32)


_GDN = lax.GatherDimensionNumbers(
    offset_dims=(), collapsed_slice_dims=(0,), start_index_map=(0,))


def _bcast_lane(vec, i):
    idx = jnp.full((16, 1), i, jnp.int32)
    return lax.gather(vec, idx, dimension_numbers=_GDN, slice_sizes=(1,),
                      mode=lax.GatherScatterMode.PROMISE_IN_BOUNDS)


def _sca_body(src_hbm, dst_hbm, et_hbm, st_hbm,
              ex_hbm, idx1_hbm, den_hbm,
              srcv, dstv, etv, idx1v, exv, idxs2, idxt2, dst2,
              sbuf, tbuf, sbuf2, tbuf2, zbuf, den_sp, sg1, sg2, sg3, sg4):
    cid = lax.axis_index("c")
    sid = lax.axis_index("s")
    wid = cid * NS + sid
    base = wid * EPT

    def zb(k, _):
        zbuf[pl.ds(k * 16, 16)] = _zero16()
        return 0
    lax.fori_loop(0, ROWS_PT // 16, zb, 0)
    pltpu.sync_copy(zbuf, den_sp.at[pl.ds(sid * ROWS_PT, ROWS_PT)])

    pltpu.sync_copy(src_hbm.at[pl.ds(base, EPT)], srcv)
    pltpu.sync_copy(dst_hbm.at[pl.ds(base, EPT)], dstv)
    pltpu.sync_copy(et_hbm.at[pl.ds(base, EPT)], etv)

    def idx_batch(b, _):
        for g in range(8):
            off = b * 128 + g * 16
            s_i = srcv[pl.ds(off, 16)]
            d_i = dstv[pl.ds(off, 16)]
            t_i = etv[pl.ds(off, 16)]
            idxs2[b, pl.ds(g * 16, 16)] = s_i * 16 + t_i * 2
            idxt2[b, pl.ds(g * 16, 16)] = d_i * 16 + t_i * 2 + 1
            dst2[b, pl.ds(g * 16, 16)] = d_i
            idx1v[pl.ds(off, 16)] = t_i * N + s_i
        return 0
    lax.fori_loop(0, NB, idx_batch, 0)

    plsc.subcore_barrier()

    def fire(b, sb, tb, g1, g2):
        pltpu.async_copy(st_hbm.at[idxs2.at[b]], sb, g1)
        pltpu.async_copy(st_hbm.at[idxt2.at[b]], tb, g2)

    def drain(sb, tb, g1, g2):
        pltpu.make_async_copy(st_hbm.at[idxs2.at[0]], sb, g1).wait()
        pltpu.make_async_copy(st_hbm.at[idxt2.at[0]], tb, g2).wait()

    def ex_batch(b, sb, tb):
        for g in range(8):
            off = b * 128 + g * 16
            sc = sb[pl.ds(g * 16, 16)] + tb[pl.ds(g * 16, 16)]
            sc = jnp.where(sc >= 0.0, sc, sc * 0.2)
            gid = base + off + lax.iota(jnp.int32, 16)
            ex = jnp.where(gid < E, jnp.exp(sc), 0.0)
            exv[pl.ds(off, 16)] = ex
        pltpu.sync_copy(exv.at[pl.ds(b * 128, 128)],
                        den_sp.at[dst2.at[b]], add=True)

    fire(0, sbuf, tbuf, sg1, sg2)

    def pair(k, _):
        b0 = 2 * k
        fire(b0 + 1, sbuf2, tbuf2, sg3, sg4)
        drain(sbuf, tbuf, sg1, sg2)
        ex_batch(b0, sbuf, tbuf)

        @pl.when(k < NB // 2 - 1)
        def _():
            fire(b0 + 2, sbuf, tbuf, sg1, sg2)
        drain(sbuf2, tbuf2, sg3, sg4)
        ex_batch(b0 + 1, sbuf2, tbuf2)
        return 0
    lax.fori_loop(0, NB // 2, pair, 0)

    pltpu.sync_copy(exv, ex_hbm.at[pl.ds(base, EPT)])
    pltpu.sync_copy(idx1v, idx1_hbm.at[pl.ds(base, EPT)])

    plsc.subcore_barrier()
    pltpu.sync_copy(den_sp.at[pl.ds(sid * ROWS_PT, ROWS_PT)],
                    den_hbm.at[pl.ds(cid * N_PAD + sid * ROWS_PT, ROWS_PT)])


def _sca(src, dst, et, st_flat):
    f = pl.kernel(
        _sca_body,
        out_type=(
            jax.ShapeDtypeStruct((E_PAD,), jnp.float32),
            jax.ShapeDtypeStruct((E_PAD,), jnp.int32),
            jax.ShapeDtypeStruct((2 * N_PAD,), jnp.float32),
        ),
        mesh=_mesh,
        scratch_types=[
            pltpu.VMEM((EPT,), jnp.int32),
            pltpu.VMEM((EPT,), jnp.int32),
            pltpu.VMEM((EPT,), jnp.int32),
            pltpu.VMEM((EPT,), jnp.int32),
            pltpu.VMEM((EPT,), jnp.float32),
            pltpu.VMEM((NB, 128), jnp.int32),
            pltpu.VMEM((NB, 128), jnp.int32),
            pltpu.VMEM((NB, 128), jnp.int32),
            pltpu.VMEM((128,), jnp.float32),
            pltpu.VMEM((128,), jnp.float32),
            pltpu.VMEM((128,), jnp.float32),
            pltpu.VMEM((128,), jnp.float32),
            pltpu.VMEM((ROWS_PT,), jnp.float32),
            pltpu.VMEM_SHARED((N_PAD,), jnp.float32),
            pltpu.SemaphoreType.DMA,
            pltpu.SemaphoreType.DMA,
            pltpu.SemaphoreType.DMA,
            pltpu.SemaphoreType.DMA,
        ],
    )
    return f(src, dst, et, st_flat)


RB = 128
NB2 = EPT // RB
SCALE_CHUNKS = 7


def _scb_body(idx1_hbm, dst_hbm, ex_hbm, den_hbm, z_hbm,
              outp_hbm,
              idx12, dst2, dbufA, dbufB, ebufA, ebufB, rbufA, rbufB,
              out_sp, den_sp2, semA, semB, semDA, semDB, semEA, semEB):
    cid = lax.axis_index("c")
    sid = lax.axis_index("s")
    wid = cid * NS + sid
    base = wid * EPT

    def zr(r, _):
        for c in range(HP // 16):
            rbufA[r, pl.ds(c * 16, 16)] = _zero16()
        return 0
    lax.fori_loop(0, RB, zr, 0)
    obase = pl.multiple_of(jnp.minimum(sid * OCHUNK, NOUT - OCHUNK), 8)

    def zcp(j, _):
        pltpu.sync_copy(rbufA, out_sp.at[pl.ds(
            pl.multiple_of(obase + j * RB, 8), RB)])
        return 0
    lax.fori_loop(0, OCHUNK // RB, zcp, 0)
    pltpu.sync_copy(rbufA.at[pl.ds(0, OCHUNK % RB)],
                    out_sp.at[pl.ds(obase + (OCHUNK // RB) * RB,
                                    OCHUNK % RB)])

    def dj(j, _):
        o = sid * ROWS_PT + j * RB
        pltpu.sync_copy(den_hbm.at[pl.ds(o, RB)], dbufA)
        pltpu.sync_copy(den_hbm.at[pl.ds(N_PAD + o, RB)], ebufA)
        for q in range(RB // 16):
            dbufA[pl.ds(q * 16, 16)] = (dbufA[pl.ds(q * 16, 16)]
                                        + ebufA[pl.ds(q * 16, 16)])
        pltpu.sync_copy(dbufA, den_sp2.at[pl.ds(o, RB)])
        return 0
    lax.fori_loop(0, ROWS_PT // RB, dj, 0)

    def ld(b, _):
        pltpu.sync_copy(idx1_hbm.at[pl.ds(base + b * RB, RB)], idx12.at[b])
        pltpu.sync_copy(dst_hbm.at[pl.ds(base + b * RB, RB)], dst2.at[b])
        return 0
    lax.fori_loop(0, NB2, ld, 0)

    plsc.subcore_barrier()

    def process(b, rbuf, dbuf, ebuf):
        def scale_group(g, _):
            den16 = dbuf[pl.ds(g * 16, 16)]
            al = ebuf[pl.ds(g * 16, 16)] / (den16 + 1e-16)
            for i in range(16):
                bc = _bcast_lane(al, i)
                row = g * 16 + i
                for c in range(SCALE_CHUNKS):
                    rbuf[row, pl.ds(c * 16, 16)] = (
                        rbuf[row, pl.ds(c * 16, 16)] * bc)
            return 0
        lax.fori_loop(0, RB // 16, scale_group, 0)
        pltpu.sync_copy(rbuf, out_sp.at[dst2.at[b]], add=True)

    def fire(b, rbuf, dbuf, ebuf, sr, sd, se):
        pltpu.async_copy(z_hbm.at[idx12.at[b]], rbuf, sr)
        pltpu.async_copy(den_sp2.at[dst2.at[b]], dbuf, sd)
        pltpu.async_copy(ex_hbm.at[pl.ds(base + b * RB, RB)], ebuf, se)

    def drain(rbuf, dbuf, ebuf, sr, sd, se):
        pltpu.make_async_copy(z_hbm.at[idx12.at[0]], rbuf, sr).wait()
        pltpu.make_async_copy(den_sp2.at[dst2.at[0]], dbuf, sd).wait()
        pltpu.make_async_copy(ex_hbm.at[pl.ds(base, RB)], ebuf, se).wait()

    fire(0, rbufA, dbufA, ebufA, semA, semDA, semEA)

    def pair(k, _):
        b0 = 2 * k
        fire(b0 + 1, rbufB, dbufB, ebufB, semB, semDB, semEB)
        drain(rbufA, dbufA, ebufA, semA, semDA, semEA)
        process(b0, rbufA, dbufA, ebufA)

        @pl.when(k < NB2 // 2 - 1)
        def _():
            fire(b0 + 2, rbufA, dbufA, ebufA, semA, semDA, semEA)
        drain(rbufB, dbufB, ebufB, semB, semDB, semEB)
        process(b0 + 1, rbufB, dbufB, ebufB)
        return 0
    lax.fori_loop(0, NB2 // 2, pair, 0)

    plsc.subcore_barrier()
    pltpu.sync_copy(out_sp.at[pl.ds(obase, OCHUNK)],
                    outp_hbm.at[cid, pl.ds(obase, OCHUNK)])


def _scb(idx1, dst, ex, den, z_flat):
    f = pl.kernel(
        _scb_body,
        out_type=jax.ShapeDtypeStruct((2, NOUT, HP), jnp.float32),
        mesh=_mesh,
        scratch_types=[
            pltpu.VMEM((NB2, RB), jnp.int32),
            pltpu.VMEM((NB2, RB), jnp.int32),
            pltpu.VMEM((RB,), jnp.float32),
            pltpu.VMEM((RB,), jnp.float32),
            pltpu.VMEM((RB,), jnp.float32),
            pltpu.VMEM((RB,), jnp.float32),
            pltpu.VMEM((RB, HP), jnp.float32),
            pltpu.VMEM((RB, HP), jnp.float32),
            pltpu.VMEM_SHARED((NOUT, HP), jnp.float32),
            pltpu.VMEM_SHARED((N_PAD,), jnp.float32),
            pltpu.SemaphoreType.DMA,
            pltpu.SemaphoreType.DMA,
            pltpu.SemaphoreType.DMA,
            pltpu.SemaphoreType.DMA,
            pltpu.SemaphoreType.DMA,
            pltpu.SemaphoreType.DMA,
        ],
    )
    return f(idx1, dst, ex, den, z_flat)


def kernel(x, edge_index, edge_type, W_fc, b_fc, W1, a1_src, a1_dst,
           W2, a2_src, a2_dst, Wc1, bc1, Wc2, bc2):
    f32 = jnp.float32
    src = jnp.pad(edge_index[0].astype(jnp.int32), (0, E_PAD - E))
    dst = jnp.pad(edge_index[1].astype(jnp.int32), (0, E_PAD - E))
    et = jnp.pad(edge_type.astype(jnp.int32), (0, E_PAD - E))

    w1p = jnp.pad(W1.astype(f32), ((0, 0), (0, 0), (0, HP - H)))
    a1p = jnp.pad(jnp.stack([a1_src, a1_dst], axis=-1).astype(f32),
                  ((0, HP - H), (0, 0)))
    a1p = jnp.broadcast_to(a1p[None], (R, HP, 2))
    w2p = jnp.pad(W2.astype(f32), ((0, 0), (0, HP - H), (0, HP - H)))
    a2p = jnp.pad(jnp.stack([a2_src, a2_dst], axis=-1).astype(f32),
                  ((0, HP - H), (0, 0)))
    a2p = jnp.broadcast_to(a2p[None], (R, HP, 2))
    wca = Wc1[:G_DIM].astype(f32)
    wcb = jnp.pad(Wc1[G_DIM:].astype(f32), ((0, HP - H), (0, 0)))
    wc2p = jnp.pad(Wc2.astype(f32), ((0, 0), (0, 8 - TAG)))
    bc2p = jnp.pad(bc2.astype(f32), (0, 8 - TAG)).reshape(1, 8)

    h, z1, st1 = _tc1(x.astype(f32), W_fc.astype(f32),
                      b_fc.astype(f32).reshape(1, G_DIM), w1p, a1p)
    ex1, idx1a, den1 = _sca(src, dst, et, st1.reshape(-1))
    p1 = _scb(idx1a, dst, ex1, den1, z1.reshape(R * N, HP))

    z2, st2 = _tc2(p1, w2p, a2p)
    ex2, idx2a, den2 = _sca(src, dst, et, st2.reshape(-1))
    p2 = _scb(idx2a, dst, ex2, den2, z2.reshape(R * N, HP))

    (out8,) = _tc3(h, p2, wca, wcb, bc1.astype(f32).reshape(1, HC), wc2p, bc2p)
    return out8[:, :TAG]
